# baseline (speedup 1.0000x reference)
import functools

import jax
import jax.numpy as jnp
from jax import lax
from jax.experimental import pallas as pl
from jax.experimental.pallas import tpu as pltpu
from jax.experimental.pallas import tpu_sc as plsc

D_IN = 128
D_OUT = 32
NC = 2
NS = 16
L = 16
NW = NC * NS
CH = 128
NB = 3
CHL = 576
NB2 = 2
CHS = 3 * CH
ETQ = 1152
BIGF = 3.0e38
_SCP = pltpu.CompilerParams(use_tc_tiling_on_sc=False,
                            needs_layout_passes=False)


def _rsqrt_nr(v):
    i = lax.bitcast_convert_type(v, jnp.int32)
    i = 0x5F3759DF - lax.shift_right_logical(i, 1)
    y = lax.bitcast_convert_type(i, jnp.float32)
    for _ in range(3):
        y = y * (1.5 - 0.5 * v * y * y)
    return y


def _silu(z):
    return z / (1.0 + jnp.exp(-z))


def _make_mesh():
    return plsc.VectorSubcoreMesh(core_axis_name="c", subcore_axis_name="s",
                                  num_cores=NC, num_subcores=NS)


def _t1_body(x_ref, pos_ref, wx_ref, wr_ref, a_ref, b_ref):
    b = jnp.dot(pos_ref[...], wr_ref[...], preferred_element_type=jnp.float32)
    a = jnp.dot(x_ref[...], wx_ref[...], preferred_element_type=jnp.float32) + b
    a_ref[...] = a
    b_ref[...] = b


def _t4_body(u_ref, d_ref, o_ref, *, n):
    u = u_ref[0, 0:n, :] + u_ref[1, 0:n, :]
    d = d_ref[0, 0:n] + d_ref[1, 0:n]
    o_ref[...] = u / (d[:, None] + 1e-16)


def _build_s1(e_pad, et, cpt):
    mesh = _make_mesh()

    @functools.partial(
        pl.kernel,
        out_type=(
            jax.ShapeDtypeStruct((e_pad, D_OUT), jnp.float32),
            jax.ShapeDtypeStruct((NW, 4 * L), jnp.float32),
        ),
        mesh=mesh,
        compiler_params=_SCP,
        scratch_types=(
            [pltpu.VMEM((cpt, CH), jnp.int32)] * 2
            + [pltpu.VMEM((CH, D_OUT), jnp.float32)] * (3 * NB)
            + [pltpu.VMEM((4 * L,), jnp.float32)]
            + [pltpu.SemaphoreType.DMA] * (3 * NB)
        ),
    )
    def s1(a_t, b_t, src_g, dst_g, l_g, part_g, *refs):
        idxs2, idxd2 = refs[0], refs[1]
        ras = refs[2:2 + NB]
        rbs = refs[2 + NB:2 + 2 * NB]
        lbs = refs[2 + 2 * NB:2 + 3 * NB]
        pbuf = refs[2 + 3 * NB]
        sas = refs[3 + 3 * NB:3 + 4 * NB]
        sbs = refs[3 + 4 * NB:3 + 5 * NB]
        sls = refs[3 + 5 * NB:3 + 6 * NB]

        cid = lax.axis_index("c")
        sid = lax.axis_index("s")
        wid = sid * NC + cid
        base = wid * et

        pltpu.sync_copy(src_g.at[wid], idxs2)
        pltpu.sync_copy(dst_g.at[wid], idxd2)

        def gissue(c, b):
            pltpu.async_copy(a_t.at[idxs2.at[c]], ras[b], sas[b])
            pltpu.async_copy(b_t.at[idxd2.at[c]], rbs[b], sbs[b])

        for b in range(NB):
            gissue(b, b)

        def outer(cg, carry):
            car = carry
            for b in range(NB):
                c = cg * NB + b
                pltpu.make_async_copy(a_t.at[idxs2.at[c]], ras[b],
                                      sas[b]).wait()
                pltpu.make_async_copy(b_t.at[idxd2.at[c]], rbs[b],
                                      sbs[b]).wait()

                @pl.when(cg > 0)
                def _():
                    pltpu.make_async_copy(
                        lbs[b], l_g.at[pl.ds(base + (c - NB) * CH, CH)],
                        sls[b]).wait()

                ra, rb, lb = ras[b], rbs[b], lbs[b]

                def edge(e, car2):
                    s0, s1v, q0, q1 = car2
                    l0 = ra[e, pl.ds(0, L)] - rb[e, pl.ds(0, L)]
                    l1 = ra[e, pl.ds(L, L)] - rb[e, pl.ds(L, L)]
                    lb[e, pl.ds(0, L)] = l0
                    lb[e, pl.ds(L, L)] = l1
                    return (s0 + l0, s1v + l1, q0 + l0 * l0, q1 + l1 * l1)

                car = lax.fori_loop(0, CH, edge, car, unroll=4)
                pltpu.async_copy(lbs[b], l_g.at[pl.ds(base + c * CH, CH)],
                                 sls[b])

                @pl.when(c + NB < cpt)
                def _():
                    gissue(c + NB, b)
            return car

        z = jnp.zeros((L,), jnp.float32)
        s0, s1v, q0, q1 = lax.fori_loop(0, cpt // NB, outer, (z, z, z, z))
        for b in range(NB):
            c = cpt - NB + b
            pltpu.make_async_copy(lbs[b], l_g.at[pl.ds(base + c * CH, CH)],
                                  sls[b]).wait()
        pbuf[pl.ds(0, L)] = s0
        pbuf[pl.ds(L, L)] = s1v
        pbuf[pl.ds(2 * L, L)] = q0
        pbuf[pl.ds(3 * L, L)] = q1
        pltpu.sync_copy(pbuf, part_g.at[wid])

    return s1


def _build_s2(e_pad, et, cpt2, e_tot):
    mesh = _make_mesh()
    inv_e = 1.0 / float(e_tot)

    @functools.partial(
        pl.kernel,
        out_type=(
            jax.ShapeDtypeStruct((e_pad,), jnp.float32),
            jax.ShapeDtypeStruct((NW, L), jnp.float32),
        ),
        mesh=mesh,
        compiler_params=_SCP,
        scratch_types=(
            [pltpu.VMEM((NW, 4 * L), jnp.float32),
             pltpu.VMEM((4 * L,), jnp.float32),
             pltpu.VMEM((2 * L,), jnp.float32),
             pltpu.VMEM((L,), jnp.float32)]
            + [pltpu.VMEM((CHL, D_OUT), jnp.float32)] * NB2
            + [pltpu.VMEM((CHL,), jnp.float32)] * NB2
            + [pltpu.SemaphoreType.DMA] * (2 * NB2)
        ),
    )
    def s2(part_g, l_g, wg_g, prm_g, g_g, part2_g, *refs):
        pall, prmb, wgb, p2buf = refs[0], refs[1], refs[2], refs[3]
        lbs = refs[4:4 + NB2]
        gbs = refs[4 + NB2:4 + 2 * NB2]
        sls = refs[4 + 2 * NB2:4 + 3 * NB2]
        sgs = refs[4 + 3 * NB2:4 + 4 * NB2]

        cid = lax.axis_index("c")
        sid = lax.axis_index("s")
        wid = sid * NC + cid
        base = wid * et

        for b in range(NB2):
            pltpu.async_copy(l_g.at[pl.ds(base + b * CHL, CHL)], lbs[b],
                             sls[b])

        pltpu.sync_copy(part_g, pall)
        pltpu.sync_copy(prm_g, prmb)
        pltpu.sync_copy(wg_g, wgb)

        def red(r, car):
            c0, c1, c2, c3 = car
            return (c0 + pall[r, pl.ds(0, L)],
                    c1 + pall[r, pl.ds(L, L)],
                    c2 + pall[r, pl.ds(2 * L, L)],
                    c3 + pall[r, pl.ds(3 * L, L)])

        z = jnp.zeros((L,), jnp.float32)
        sl0, sl1, sq0, sq1 = lax.fori_loop(0, NW, red, (z, z, z, z))
        m0 = sl0 * inv_e
        m1 = sl1 * inv_e
        v0 = sq0 * inv_e - m0 * m0
        v1 = sq1 * inv_e - m1 * m1
        r0 = _rsqrt_nr(v0 + 1e-5)
        r1 = _rsqrt_nr(v1 + 1e-5)
        s1a = prmb[pl.ds(0, L)] * r0
        s1b = prmb[pl.ds(L, L)] * r1
        t1a = prmb[pl.ds(2 * L, L)] - m0 * s1a
        t1b = prmb[pl.ds(3 * L, L)] - m1 * s1b
        li = lax.iota(jnp.int32, L)
        wglo = wgb[pl.ds(0, L)]
        wghi = wgb[pl.ds(L, L)]

        def outer(cg, carry):
            car = carry
            for b in range(NB2):
                c = cg * NB2 + b
                be = base + c * CHL
                pltpu.make_async_copy(l_g.at[pl.ds(be, CHL)], lbs[b],
                                      sls[b]).wait()

                @pl.when(cg > 0)
                def _():
                    pb = base + (c - NB2) * CHL
                    pltpu.make_async_copy(gbs[b], g_g.at[pl.ds(pb, CHL)],
                                          sgs[b]).wait()

                lb, gb = lbs[b], gbs[b]

                def kgroup(k, car2):
                    gs, gq, gmn, gmx = car2
                    gv = jnp.zeros((L,), jnp.float32)
                    for j in range(L):
                        e = k * L + j
                        z0 = lb[e, pl.ds(0, L)] * s1a + t1a
                        z1 = lb[e, pl.ds(L, L)] * s1b + t1b
                        h0 = _silu(z0)
                        h1 = _silu(z1)
                        s = jnp.sum(h0 * wglo + h1 * wghi)
                        gv = jnp.where(li == j, s, gv)
                    gb[pl.ds(k * L, L)] = gv
                    valid = (be + k * L + li) < e_tot
                    gz = jnp.where(valid, gv, 0.0)
                    return (gs + gz, gq + gz * gz,
                            jnp.minimum(gmn, jnp.where(valid, gv, BIGF)),
                            jnp.maximum(gmx, jnp.where(valid, gv, -BIGF)))

                car = lax.fori_loop(0, CHL // L, kgroup, car)

                @pl.when(c + NB2 < cpt2)
                def _():
                    nbo = base + (c + NB2) * CHL
                    pltpu.async_copy(l_g.at[pl.ds(nbo, CHL)], lbs[b], sls[b])
                pltpu.async_copy(gb, g_g.at[pl.ds(be, CHL)], sgs[b])
            return car

        gs, gq, gmn, gmx = lax.fori_loop(
            0, cpt2 // NB2, outer,
            (z, z, jnp.full((L,), BIGF), jnp.full((L,), -BIGF)))
        for b in range(NB2):
            pb = base + (cpt2 - NB2 + b) * CHL
            pltpu.make_async_copy(gbs[b], g_g.at[pl.ds(pb, CHL)],
                                  sgs[b]).wait()
        row = jnp.where(li == 0, jnp.sum(gs),
                        jnp.where(li == 1, jnp.sum(gq),
                                  jnp.where(li == 2, jnp.min(gmn),
                                            jnp.where(li == 3, jnp.max(gmx),
                                                      0.0))))
        p2buf[...] = row
        pltpu.sync_copy(p2buf, part2_g.at[wid])

    return s2


def _build_s3(e_pad, et, cpt, cpt3, e_tot, na):
    mesh = _make_mesh()
    inv_e = 1.0 / float(e_tot)
    rows_per_tile = na // NS
    sub = CHS // CH

    @functools.partial(
        pl.kernel,
        out_type=(
            jax.ShapeDtypeStruct((NC, na, D_OUT), jnp.float32),
            jax.ShapeDtypeStruct((NC, na), jnp.float32),
        ),
        mesh=mesh,
        compiler_params=_SCP,
        scratch_types=(
            [pltpu.VMEM((NW, L), jnp.float32),
             pltpu.VMEM((L,), jnp.float32),
             pltpu.VMEM((cpt, CH), jnp.int32),
             pltpu.VMEM((et,), jnp.float32),
             pltpu.VMEM((et,), jnp.float32),
             pltpu.VMEM((CH,), jnp.float32),
             pltpu.VMEM((NW, 4 * L), jnp.float32),
             pltpu.VMEM((4 * L,), jnp.float32)]
            + [pltpu.VMEM((CHS, D_OUT), jnp.float32)] * (2 * NB)
            + [pltpu.VMEM_SHARED((na, D_OUT), jnp.float32),
               pltpu.VMEM_SHARED((na,), jnp.float32)]
            + [pltpu.SemaphoreType.DMA] * (2 * NB + 1)
        ),
    )
    def s3(part_g, part2_g, l_g, g_g, dst_g, prm_g, prm2_g,
           u_acc_g, d_acc_g, *refs):
        p2all, prm2b, idxd2, gall, wall, zdbuf = refs[0:6]
        pall, prmb = refs[6], refs[7]
        hbs = refs[8:8 + NB]
        vbs = refs[8 + NB:8 + 2 * NB]
        uacc, dacc = refs[8 + 2 * NB], refs[9 + 2 * NB]
        shs = refs[10 + 2 * NB:10 + 3 * NB]
        sus = refs[10 + 3 * NB:10 + 4 * NB]
        sds = refs[10 + 4 * NB]

        cid = lax.axis_index("c")
        sid = lax.axis_index("s")
        wid = sid * NC + cid
        base = wid * et

        for b in range(NB):
            pltpu.async_copy(l_g.at[pl.ds(base + b * CHS, CHS)], hbs[b],
                             shs[b])
        pltpu.sync_copy(part_g, pall)
        pltpu.sync_copy(prm_g, prmb)
        pltpu.sync_copy(part2_g, p2all)
        pltpu.sync_copy(prm2_g, prm2b)
        pltpu.sync_copy(dst_g.at[wid], idxd2)
        pltpu.sync_copy(g_g.at[wid], gall)

        def red1(r, car):
            c0, c1, c2, c3 = car
            return (c0 + pall[r, pl.ds(0, L)],
                    c1 + pall[r, pl.ds(L, L)],
                    c2 + pall[r, pl.ds(2 * L, L)],
                    c3 + pall[r, pl.ds(3 * L, L)])

        z4 = jnp.zeros((L,), jnp.float32)
        sl0, sl1, sq0, sq1 = lax.fori_loop(0, NW, red1, (z4, z4, z4, z4))
        m0 = sl0 * inv_e
        m1 = sl1 * inv_e
        v0 = sq0 * inv_e - m0 * m0
        v1 = sq1 * inv_e - m1 * m1
        r0 = _rsqrt_nr(v0 + 1e-5)
        r1 = _rsqrt_nr(v1 + 1e-5)
        s1a = prmb[pl.ds(0, L)] * r0
        s1b = prmb[pl.ds(L, L)] * r1
        t1a = prmb[pl.ds(2 * L, L)] - m0 * s1a
        t1b = prmb[pl.ds(3 * L, L)] - m1 * s1b

        p2v = prm2b[pl.ds(0, L)]
        li = lax.iota(jnp.int32, L)
        is_sum = li < 2
        is_min = li == 2

        def red(r, acc_v):
            rrow = p2all[r, pl.ds(0, L)]
            return jnp.where(is_sum, acc_v + rrow,
                             jnp.where(is_min, jnp.minimum(acc_v, rrow),
                                       jnp.maximum(acc_v, rrow)))

        accv = lax.fori_loop(1, NW, red, p2all[0, pl.ds(0, L)])
        gsum = jnp.sum(jnp.where(li == 0, accv, 0.0))
        gsq = jnp.sum(jnp.where(li == 1, accv, 0.0))
        gmn = jnp.sum(jnp.where(li == 2, accv, 0.0))
        gmx = jnp.sum(jnp.where(li == 3, accv, 0.0))
        m2 = gsum * inv_e
        v2 = gsq * inv_e - m2 * m2
        rsv = _rsqrt_nr(jnp.full((L,), v2 + 1e-5))
        s2v = rsv * p2v[0]
        t2v = p2v[1] - m2 * s2v
        ze = jnp.where(li == 0, gmn, gmx) * s2v + t2v
        cmax = jnp.max(_silu(ze))
        s2s = jnp.sum(jnp.where(li == 0, s2v, 0.0))
        t2s = jnp.sum(jnp.where(li == 0, t2v, 0.0))

        def wstage(k, car):
            gv = gall[pl.ds(k * L, L)]
            gate = _silu(gv * s2s + t2s)
            wv = jnp.exp(gate - cmax)
            wall[pl.ds(k * L, L)] = jnp.where(base + k * L + li < e_tot,
                                              wv, 0.0)
            return car

        lax.fori_loop(0, et // L, wstage, 0, unroll=4)

        zv = jnp.zeros((L,), jnp.float32)
        zstage = vbs[0]

        def zrow(r, car):
            zstage[r, pl.ds(0, L)] = zv
            zstage[r, pl.ds(L, L)] = zv
            return car

        lax.fori_loop(0, CH, zrow, 0)
        for k in range(CH // L):
            zdbuf[pl.ds(k * L, L)] = zv
        for j in range(rows_per_tile // CH):
            pltpu.sync_copy(zstage.at[pl.ds(0, CH)],
                            uacc.at[pl.ds(sid * rows_per_tile + j * CH, CH)])
            pltpu.sync_copy(zdbuf,
                            dacc.at[pl.ds(sid * rows_per_tile + j * CH, CH)])
        plsc.subcore_barrier()

        def outer(cg, car):
            for b in range(NB):
                c = cg * NB + b
                be = base + c * CHS
                pltpu.make_async_copy(l_g.at[pl.ds(be, CHS)], hbs[b],
                                      shs[b]).wait()

                @pl.when(cg > 0)
                def _():
                    for j in range(sub):
                        pltpu.make_async_copy(
                            vbs[b].at[pl.ds(j * CH, CH)],
                            uacc.at[idxd2.at[(c - NB) * sub + j]],
                            sus[b]).wait()

                hb, vb = hbs[b], vbs[b]

                def group(k, car2):
                    wv = wall[pl.ds(c * CHS + k * L, L)]
                    for j in range(L):
                        e = k * L + j
                        w = wv[j]
                        z0 = hb[e, pl.ds(0, L)] * s1a + t1a
                        z1 = hb[e, pl.ds(L, L)] * s1b + t1b
                        vb[e, pl.ds(0, L)] = _silu(z0) * w
                        vb[e, pl.ds(L, L)] = _silu(z1) * w
                    return car2

                lax.fori_loop(0, CHS // L, group, 0)

                @pl.when(c + NB < cpt3)
                def _():
                    nbo = base + (c + NB) * CHS
                    pltpu.async_copy(l_g.at[pl.ds(nbo, CHS)], hbs[b], shs[b])

                for j in range(sub):
                    pltpu.async_copy(
                        vb.at[pl.ds(j * CH, CH)],
                        uacc.at[idxd2.at[c * sub + j]],
                        sus[b], add=True)
                    pltpu.async_copy(
                        wall.at[pl.ds(c * CHS + j * CH, CH)],
                        dacc.at[idxd2.at[c * sub + j]],
                        sds, add=True)
            return car

        lax.fori_loop(0, cpt3 // NB, outer, 0)
        for b in range(NB):
            c = cpt3 - NB + b
            for j in range(sub):
                pltpu.make_async_copy(
                    vbs[b].at[pl.ds(j * CH, CH)],
                    uacc.at[idxd2.at[c * sub + j]],
                    sus[b]).wait()
        for c in range(cpt3):
            for j in range(sub):
                pltpu.make_async_copy(
                    wall.at[pl.ds(c * CHS + j * CH, CH)],
                    dacc.at[idxd2.at[c * sub + j]],
                    sds).wait()
        plsc.subcore_barrier()
        pltpu.sync_copy(
            uacc.at[pl.ds(sid * rows_per_tile, rows_per_tile)],
            u_acc_g.at[cid, pl.ds(sid * rows_per_tile, rows_per_tile)])
        pltpu.sync_copy(
            dacc.at[pl.ds(sid * rows_per_tile, rows_per_tile)],
            d_acc_g.at[cid, pl.ds(sid * rows_per_tile, rows_per_tile)])

    return s3


def kernel(x, pos, edge_index, W_local, b_local, bn_local_gamma,
           bn_local_beta, W_gate, b_gate, bn_gate_gamma, bn_gate_beta):
    n = x.shape[0]
    e_tot = edge_index.shape[1] + n
    et = -(-e_tot // (NW * ETQ)) * ETQ
    cpt = et // CH
    cpt2 = et // CHL
    cpt3 = et // CHS
    e_pad = et * NW
    np_rows = ((n + 16) + 7) // 8 * 8
    na = -(-np_rows // (NS * CH)) * NS * CH

    idt = edge_index.dtype
    loop = jnp.arange(n, dtype=idt)
    pad = e_pad - e_tot
    src = jnp.concatenate([edge_index[0], loop, jnp.full((pad,), n, idt)])
    dst = jnp.concatenate([edge_index[1], loop, jnp.full((pad,), n, idt)])
    src3 = src.reshape(NW, cpt, CH)
    dst3 = dst.reshape(NW, cpt, CH)
    xp = jnp.pad(x, ((0, np_rows - n), (0, 0)))
    posp = jnp.pad(pos, ((0, np_rows - n), (0, 5)))
    wx = W_local[:D_IN]
    wr = jnp.pad(W_local[D_IN:], ((0, 8 - (W_local.shape[0] - D_IN)), (0, 0)))
    prm1 = jnp.concatenate([bn_local_gamma, bn_local_beta])
    wg = W_gate[:, 0]
    prm2 = jnp.concatenate(
        [bn_gate_gamma, bn_gate_beta, jnp.zeros((14,), jnp.float32)])

    a_t, b_t = pl.pallas_call(
        _t1_body,
        out_shape=(
            jax.ShapeDtypeStruct((np_rows, D_OUT), jnp.float32),
            jax.ShapeDtypeStruct((np_rows, D_OUT), jnp.float32),
        ),
    )(xp, posp, wx, wr)

    l_g, part = _build_s1(e_pad, et, cpt)(a_t, b_t, src3, dst3)
    g_g, part2 = _build_s2(e_pad, et, cpt2, e_tot)(part, l_g, wg, prm1)
    g2 = g_g.reshape(NW, et)
    u_p, d_p = _build_s3(e_pad, et, cpt, cpt3, e_tot, na)(
        part, part2, l_g, g2, dst3, prm1, prm2)

    out = pl.pallas_call(
        functools.partial(_t4_body, n=n),
        out_shape=jax.ShapeDtypeStruct((n, D_OUT), jnp.float32),
    )(u_p, d_p)
    return out


# speedup vs baseline: 1.0786x; 1.0786x over previous
import functools

import jax
import jax.numpy as jnp
from jax import lax
from jax.experimental import pallas as pl
from jax.experimental.pallas import tpu as pltpu
from jax.experimental.pallas import tpu_sc as plsc

D_IN = 128
D_OUT = 32
NC = 2
NS = 16
L = 16
NW = NC * NS
CH = 128
NB = 3
CHL = 576
NB2 = 2
CHS = 3 * CH
ETQ = 1152
BIGF = 3.0e38
_SCP = pltpu.CompilerParams(use_tc_tiling_on_sc=False,
                            needs_layout_passes=False)


def _rsqrt_nr(v):
    i = lax.bitcast_convert_type(v, jnp.int32)
    i = 0x5F3759DF - lax.shift_right_logical(i, 1)
    y = lax.bitcast_convert_type(i, jnp.float32)
    for _ in range(3):
        y = y * (1.5 - 0.5 * v * y * y)
    return y


def _silu(z):
    return z / (1.0 + jnp.exp(-z))


def _make_mesh():
    return plsc.VectorSubcoreMesh(core_axis_name="c", subcore_axis_name="s",
                                  num_cores=NC, num_subcores=NS)


def _t1_body(x_ref, pos_ref, wx_ref, wr_ref, a_ref, b_ref):
    b = jnp.dot(pos_ref[...], wr_ref[...], preferred_element_type=jnp.float32)
    a = jnp.dot(x_ref[...], wx_ref[...], preferred_element_type=jnp.float32) + b
    a_ref[...] = a
    b_ref[...] = b


def _t4_body(u_ref, d_ref, o_ref, *, n):
    u = u_ref[0, 0:n, :] + u_ref[1, 0:n, :]
    d = d_ref[0, 0:n] + d_ref[1, 0:n]
    o_ref[...] = u / (d[:, None] + 1e-16)


def _build_s1(e_pad, et, cpt):
    mesh = _make_mesh()

    @functools.partial(
        pl.kernel,
        out_type=(
            jax.ShapeDtypeStruct((e_pad, D_OUT), jnp.float32),
            jax.ShapeDtypeStruct((NW, 4 * L), jnp.float32),
        ),
        mesh=mesh,
        compiler_params=_SCP,
        scratch_types=(
            [pltpu.VMEM((cpt, CH), jnp.int32)] * 2
            + [pltpu.VMEM((CH, D_OUT), jnp.float32)] * (3 * NB)
            + [pltpu.VMEM((4 * L,), jnp.float32)]
            + [pltpu.SemaphoreType.DMA] * (3 * NB)
        ),
    )
    def s1(a_t, b_t, src_g, dst_g, l_g, part_g, *refs):
        idxs2, idxd2 = refs[0], refs[1]
        ras = refs[2:2 + NB]
        rbs = refs[2 + NB:2 + 2 * NB]
        lbs = refs[2 + 2 * NB:2 + 3 * NB]
        pbuf = refs[2 + 3 * NB]
        sas = refs[3 + 3 * NB:3 + 4 * NB]
        sbs = refs[3 + 4 * NB:3 + 5 * NB]
        sls = refs[3 + 5 * NB:3 + 6 * NB]

        cid = lax.axis_index("c")
        sid = lax.axis_index("s")
        wid = sid * NC + cid
        base = wid * et

        pltpu.sync_copy(src_g.at[wid], idxs2)
        pltpu.sync_copy(dst_g.at[wid], idxd2)

        def gissue(c, b):
            pltpu.async_copy(a_t.at[idxs2.at[c]], ras[b], sas[b])
            pltpu.async_copy(b_t.at[idxd2.at[c]], rbs[b], sbs[b])

        for b in range(NB):
            gissue(b, b)

        def outer(cg, carry):
            car = carry
            for b in range(NB):
                c = cg * NB + b
                pltpu.make_async_copy(a_t.at[idxs2.at[c]], ras[b],
                                      sas[b]).wait()
                pltpu.make_async_copy(b_t.at[idxd2.at[c]], rbs[b],
                                      sbs[b]).wait()

                @pl.when(cg > 0)
                def _():
                    pltpu.make_async_copy(
                        lbs[b], l_g.at[pl.ds(base + (c - NB) * CH, CH)],
                        sls[b]).wait()

                ra, rb, lb = ras[b], rbs[b], lbs[b]

                def edge(e, car2):
                    s0, s1v, q0, q1 = car2
                    l0 = ra[e, pl.ds(0, L)] - rb[e, pl.ds(0, L)]
                    l1 = ra[e, pl.ds(L, L)] - rb[e, pl.ds(L, L)]
                    lb[e, pl.ds(0, L)] = l0
                    lb[e, pl.ds(L, L)] = l1
                    return (s0 + l0, s1v + l1, q0 + l0 * l0, q1 + l1 * l1)

                car = lax.fori_loop(0, CH, edge, car, unroll=4)
                pltpu.async_copy(lbs[b], l_g.at[pl.ds(base + c * CH, CH)],
                                 sls[b])

                @pl.when(c + NB < cpt)
                def _():
                    gissue(c + NB, b)
            return car

        z = jnp.zeros((L,), jnp.float32)
        s0, s1v, q0, q1 = lax.fori_loop(0, cpt // NB, outer, (z, z, z, z))
        for b in range(NB):
            c = cpt - NB + b
            pltpu.make_async_copy(lbs[b], l_g.at[pl.ds(base + c * CH, CH)],
                                  sls[b]).wait()
        pbuf[pl.ds(0, L)] = s0
        pbuf[pl.ds(L, L)] = s1v
        pbuf[pl.ds(2 * L, L)] = q0
        pbuf[pl.ds(3 * L, L)] = q1
        pltpu.sync_copy(pbuf, part_g.at[wid])

    return s1


def _build_s2(e_pad, et, cpt2, e_tot):
    mesh = _make_mesh()
    inv_e = 1.0 / float(e_tot)

    @functools.partial(
        pl.kernel,
        out_type=(
            jax.ShapeDtypeStruct((e_pad, D_OUT), jnp.float32),
            jax.ShapeDtypeStruct((e_pad,), jnp.float32),
            jax.ShapeDtypeStruct((NW, L), jnp.float32),
        ),
        mesh=mesh,
        compiler_params=_SCP,
        scratch_types=(
            [pltpu.VMEM((NW, 4 * L), jnp.float32),
             pltpu.VMEM((4 * L,), jnp.float32),
             pltpu.VMEM((2 * L,), jnp.float32),
             pltpu.VMEM((L,), jnp.float32)]
            + [pltpu.VMEM((CHL, D_OUT), jnp.float32)] * (2 * NB2)
            + [pltpu.VMEM((CHL,), jnp.float32)] * NB2
            + [pltpu.SemaphoreType.DMA] * (3 * NB2)
        ),
    )
    def s2(part_g, l_g, wg_g, prm_g, h_g, g_g, part2_g, *refs):
        pall, prmb, wgb, p2buf = refs[0], refs[1], refs[2], refs[3]
        lbs = refs[4:4 + NB2]
        hbs = refs[4 + NB2:4 + 2 * NB2]
        gbs = refs[4 + 2 * NB2:4 + 3 * NB2]
        sls = refs[4 + 3 * NB2:4 + 4 * NB2]
        shs = refs[4 + 4 * NB2:4 + 5 * NB2]
        sgs = refs[4 + 5 * NB2:4 + 6 * NB2]

        cid = lax.axis_index("c")
        sid = lax.axis_index("s")
        wid = sid * NC + cid
        base = wid * et

        for b in range(NB2):
            pltpu.async_copy(l_g.at[pl.ds(base + b * CHL, CHL)], lbs[b],
                             sls[b])

        pltpu.sync_copy(part_g, pall)
        pltpu.sync_copy(prm_g, prmb)
        pltpu.sync_copy(wg_g, wgb)

        def red(r, car):
            c0, c1, c2, c3 = car
            return (c0 + pall[r, pl.ds(0, L)],
                    c1 + pall[r, pl.ds(L, L)],
                    c2 + pall[r, pl.ds(2 * L, L)],
                    c3 + pall[r, pl.ds(3 * L, L)])

        z = jnp.zeros((L,), jnp.float32)
        sl0, sl1, sq0, sq1 = lax.fori_loop(0, NW, red, (z, z, z, z))
        m0 = sl0 * inv_e
        m1 = sl1 * inv_e
        v0 = sq0 * inv_e - m0 * m0
        v1 = sq1 * inv_e - m1 * m1
        r0 = _rsqrt_nr(v0 + 1e-5)
        r1 = _rsqrt_nr(v1 + 1e-5)
        s1a = prmb[pl.ds(0, L)] * r0
        s1b = prmb[pl.ds(L, L)] * r1
        t1a = prmb[pl.ds(2 * L, L)] - m0 * s1a
        t1b = prmb[pl.ds(3 * L, L)] - m1 * s1b
        li = lax.iota(jnp.int32, L)
        wglo = wgb[pl.ds(0, L)]
        wghi = wgb[pl.ds(L, L)]

        def outer(cg, carry):
            car = carry
            for b in range(NB2):
                c = cg * NB2 + b
                be = base + c * CHL
                pltpu.make_async_copy(l_g.at[pl.ds(be, CHL)], lbs[b],
                                      sls[b]).wait()

                @pl.when(cg > 0)
                def _():
                    pb = base + (c - NB2) * CHL
                    pltpu.make_async_copy(hbs[b], h_g.at[pl.ds(pb, CHL)],
                                          shs[b]).wait()
                    pltpu.make_async_copy(gbs[b], g_g.at[pl.ds(pb, CHL)],
                                          sgs[b]).wait()

                lb, hb, gb = lbs[b], hbs[b], gbs[b]

                def kgroup(k, car2):
                    gs, gq, gmn, gmx = car2
                    gv = jnp.zeros((L,), jnp.float32)
                    for j in range(L):
                        e = k * L + j
                        z0 = lb[e, pl.ds(0, L)] * s1a + t1a
                        z1 = lb[e, pl.ds(L, L)] * s1b + t1b
                        h0 = _silu(z0)
                        h1 = _silu(z1)
                        hb[e, pl.ds(0, L)] = h0
                        hb[e, pl.ds(L, L)] = h1
                        s = jnp.sum(h0 * wglo + h1 * wghi)
                        gv = jnp.where(li == j, s, gv)
                    gb[pl.ds(k * L, L)] = gv
                    valid = (be + k * L + li) < e_tot
                    gz = jnp.where(valid, gv, 0.0)
                    return (gs + gz, gq + gz * gz,
                            jnp.minimum(gmn, jnp.where(valid, gv, BIGF)),
                            jnp.maximum(gmx, jnp.where(valid, gv, -BIGF)))

                car = lax.fori_loop(0, CHL // L, kgroup, car)

                @pl.when(c + NB2 < cpt2)
                def _():
                    nbo = base + (c + NB2) * CHL
                    pltpu.async_copy(l_g.at[pl.ds(nbo, CHL)], lbs[b], sls[b])
                pltpu.async_copy(hb, h_g.at[pl.ds(be, CHL)], shs[b])
                pltpu.async_copy(gb, g_g.at[pl.ds(be, CHL)], sgs[b])
            return car

        gs, gq, gmn, gmx = lax.fori_loop(
            0, cpt2 // NB2, outer,
            (z, z, jnp.full((L,), BIGF), jnp.full((L,), -BIGF)))
        for b in range(NB2):
            pb = base + (cpt2 - NB2 + b) * CHL
            pltpu.make_async_copy(hbs[b], h_g.at[pl.ds(pb, CHL)],
                                  shs[b]).wait()
            pltpu.make_async_copy(gbs[b], g_g.at[pl.ds(pb, CHL)],
                                  sgs[b]).wait()
        row = jnp.where(li == 0, jnp.sum(gs),
                        jnp.where(li == 1, jnp.sum(gq),
                                  jnp.where(li == 2, jnp.min(gmn),
                                            jnp.where(li == 3, jnp.max(gmx),
                                                      0.0))))
        p2buf[...] = row
        pltpu.sync_copy(p2buf, part2_g.at[wid])

    return s2


def _build_s3(e_pad, et, cpt, cpt3, e_tot, na):
    mesh = _make_mesh()
    inv_e = 1.0 / float(e_tot)
    rows_per_tile = na // NS
    sub = CHS // CH

    @functools.partial(
        pl.kernel,
        out_type=(
            jax.ShapeDtypeStruct((NC, na, D_OUT), jnp.float32),
            jax.ShapeDtypeStruct((NC, na), jnp.float32),
        ),
        mesh=mesh,
        compiler_params=_SCP,
        scratch_types=(
            [pltpu.VMEM((NW, L), jnp.float32),
             pltpu.VMEM((L,), jnp.float32),
             pltpu.VMEM((cpt, CH), jnp.int32),
             pltpu.VMEM((et,), jnp.float32),
             pltpu.VMEM((et,), jnp.float32),
             pltpu.VMEM((CH,), jnp.float32)]
            + [pltpu.VMEM((CHS, D_OUT), jnp.float32)] * (2 * NB)
            + [pltpu.VMEM_SHARED((na, D_OUT), jnp.float32),
               pltpu.VMEM_SHARED((na,), jnp.float32)]
            + [pltpu.SemaphoreType.DMA] * (2 * NB + 1)
        ),
    )
    def s3(part2_g, h_g, g_g, dst_g, prm2_g, u_acc_g, d_acc_g, *refs):
        p2all, prm2b, idxd2, gall, wall, zdbuf = refs[0:6]
        hbs = refs[6:6 + NB]
        vbs = refs[6 + NB:6 + 2 * NB]
        uacc, dacc = refs[6 + 2 * NB], refs[7 + 2 * NB]
        shs = refs[8 + 2 * NB:8 + 3 * NB]
        sus = refs[8 + 3 * NB:8 + 4 * NB]
        sds = refs[8 + 4 * NB]

        cid = lax.axis_index("c")
        sid = lax.axis_index("s")
        wid = sid * NC + cid
        base = wid * et

        for b in range(NB):
            pltpu.async_copy(h_g.at[pl.ds(base + b * CHS, CHS)], hbs[b],
                             shs[b])
        pltpu.sync_copy(part2_g, p2all)
        pltpu.sync_copy(prm2_g, prm2b)
        pltpu.sync_copy(dst_g.at[wid], idxd2)
        pltpu.sync_copy(g_g.at[wid], gall)

        p2v = prm2b[pl.ds(0, L)]
        li = lax.iota(jnp.int32, L)
        is_sum = li < 2
        is_min = li == 2

        def red(r, acc_v):
            rrow = p2all[r, pl.ds(0, L)]
            return jnp.where(is_sum, acc_v + rrow,
                             jnp.where(is_min, jnp.minimum(acc_v, rrow),
                                       jnp.maximum(acc_v, rrow)))

        accv = lax.fori_loop(1, NW, red, p2all[0, pl.ds(0, L)])
        gsum = jnp.sum(jnp.where(li == 0, accv, 0.0))
        gsq = jnp.sum(jnp.where(li == 1, accv, 0.0))
        gmn = jnp.sum(jnp.where(li == 2, accv, 0.0))
        gmx = jnp.sum(jnp.where(li == 3, accv, 0.0))
        m2 = gsum * inv_e
        v2 = gsq * inv_e - m2 * m2
        rsv = _rsqrt_nr(jnp.full((L,), v2 + 1e-5))
        s2v = rsv * p2v[0]
        t2v = p2v[1] - m2 * s2v
        ze = jnp.where(li == 0, gmn, gmx) * s2v + t2v
        cmax = jnp.max(_silu(ze))
        s2s = jnp.sum(jnp.where(li == 0, s2v, 0.0))
        t2s = jnp.sum(jnp.where(li == 0, t2v, 0.0))

        def wstage(k, car):
            gv = gall[pl.ds(k * L, L)]
            gate = _silu(gv * s2s + t2s)
            wv = jnp.exp(gate - cmax)
            wall[pl.ds(k * L, L)] = jnp.where(base + k * L + li < e_tot,
                                              wv, 0.0)
            return car

        lax.fori_loop(0, et // L, wstage, 0, unroll=4)

        zv = jnp.zeros((L,), jnp.float32)
        zstage = vbs[0]

        def zrow(r, car):
            zstage[r, pl.ds(0, L)] = zv
            zstage[r, pl.ds(L, L)] = zv
            return car

        lax.fori_loop(0, CH, zrow, 0)
        for k in range(CH // L):
            zdbuf[pl.ds(k * L, L)] = zv
        for j in range(rows_per_tile // CH):
            pltpu.sync_copy(zstage.at[pl.ds(0, CH)],
                            uacc.at[pl.ds(sid * rows_per_tile + j * CH, CH)])
            pltpu.sync_copy(zdbuf,
                            dacc.at[pl.ds(sid * rows_per_tile + j * CH, CH)])
        plsc.subcore_barrier()

        def outer(cg, car):
            for b in range(NB):
                c = cg * NB + b
                be = base + c * CHS
                pltpu.make_async_copy(h_g.at[pl.ds(be, CHS)], hbs[b],
                                      shs[b]).wait()

                @pl.when(cg > 0)
                def _():
                    for j in range(sub):
                        pltpu.make_async_copy(
                            vbs[b].at[pl.ds(j * CH, CH)],
                            uacc.at[idxd2.at[(c - NB) * sub + j]],
                            sus[b]).wait()

                hb, vb = hbs[b], vbs[b]

                def group(k, car2):
                    wv = wall[pl.ds(c * CHS + k * L, L)]
                    for j in range(L):
                        e = k * L + j
                        w = wv[j]
                        vb[e, pl.ds(0, L)] = hb[e, pl.ds(0, L)] * w
                        vb[e, pl.ds(L, L)] = hb[e, pl.ds(L, L)] * w
                    return car2

                lax.fori_loop(0, CHS // L, group, 0)

                @pl.when(c + NB < cpt3)
                def _():
                    nbo = base + (c + NB) * CHS
                    pltpu.async_copy(h_g.at[pl.ds(nbo, CHS)], hbs[b], shs[b])

                for j in range(sub):
                    pltpu.async_copy(
                        vb.at[pl.ds(j * CH, CH)],
                        uacc.at[idxd2.at[c * sub + j]],
                        sus[b], add=True)
                    pltpu.async_copy(
                        wall.at[pl.ds(c * CHS + j * CH, CH)],
                        dacc.at[idxd2.at[c * sub + j]],
                        sds, add=True)
            return car

        lax.fori_loop(0, cpt3 // NB, outer, 0)
        for b in range(NB):
            c = cpt3 - NB + b
            for j in range(sub):
                pltpu.make_async_copy(
                    vbs[b].at[pl.ds(j * CH, CH)],
                    uacc.at[idxd2.at[c * sub + j]],
                    sus[b]).wait()
        for c in range(cpt3):
            for j in range(sub):
                pltpu.make_async_copy(
                    wall.at[pl.ds(c * CHS + j * CH, CH)],
                    dacc.at[idxd2.at[c * sub + j]],
                    sds).wait()
        plsc.subcore_barrier()
        pltpu.sync_copy(
            uacc.at[pl.ds(sid * rows_per_tile, rows_per_tile)],
            u_acc_g.at[cid, pl.ds(sid * rows_per_tile, rows_per_tile)])
        pltpu.sync_copy(
            dacc.at[pl.ds(sid * rows_per_tile, rows_per_tile)],
            d_acc_g.at[cid, pl.ds(sid * rows_per_tile, rows_per_tile)])

    return s3


def kernel(x, pos, edge_index, W_local, b_local, bn_local_gamma,
           bn_local_beta, W_gate, b_gate, bn_gate_gamma, bn_gate_beta):
    n = x.shape[0]
    e_tot = edge_index.shape[1] + n
    et = -(-e_tot // (NW * ETQ)) * ETQ
    cpt = et // CH
    cpt2 = et // CHL
    cpt3 = et // CHS
    e_pad = et * NW
    np_rows = ((n + 16) + 7) // 8 * 8
    na = -(-np_rows // (NS * CH)) * NS * CH

    idt = edge_index.dtype
    loop = jnp.arange(n, dtype=idt)
    pad = e_pad - e_tot
    src = jnp.concatenate([edge_index[0], loop, jnp.full((pad,), n, idt)])
    dst = jnp.concatenate([edge_index[1], loop, jnp.full((pad,), n, idt)])
    src3 = src.reshape(NW, cpt, CH)
    dst3 = dst.reshape(NW, cpt, CH)
    xp = jnp.pad(x, ((0, np_rows - n), (0, 0)))
    posp = jnp.pad(pos, ((0, np_rows - n), (0, 5)))
    wx = W_local[:D_IN]
    wr = jnp.pad(W_local[D_IN:], ((0, 8 - (W_local.shape[0] - D_IN)), (0, 0)))
    prm1 = jnp.concatenate([bn_local_gamma, bn_local_beta])
    wg = W_gate[:, 0]
    prm2 = jnp.concatenate(
        [bn_gate_gamma, bn_gate_beta, jnp.zeros((14,), jnp.float32)])

    a_t, b_t = pl.pallas_call(
        _t1_body,
        out_shape=(
            jax.ShapeDtypeStruct((np_rows, D_OUT), jnp.float32),
            jax.ShapeDtypeStruct((np_rows, D_OUT), jnp.float32),
        ),
    )(xp, posp, wx, wr)

    l_g, part = _build_s1(e_pad, et, cpt)(a_t, b_t, src3, dst3)
    h_g, g_g, part2 = _build_s2(e_pad, et, cpt2, e_tot)(part, l_g, wg, prm1)
    g2 = g_g.reshape(NW, et)
    u_p, d_p = _build_s3(e_pad, et, cpt, cpt3, e_tot, na)(
        part2, h_g, g2, dst3, prm2)

    out = pl.pallas_call(
        functools.partial(_t4_body, n=n),
        out_shape=jax.ShapeDtypeStruct((n, D_OUT), jnp.float32),
    )(u_p, d_p)
    return out


# speedup vs baseline: 1.1753x; 1.0896x over previous
import functools

import jax
import jax.numpy as jnp
from jax import lax
from jax.experimental import pallas as pl
from jax.experimental.pallas import tpu as pltpu
from jax.experimental.pallas import tpu_sc as plsc

D_IN = 128
D_OUT = 32
NC = 2
NS = 16
L = 16
NW = NC * NS
CH = 128
NB = 3
CHL = 576
NB2 = 2
CHS = 3 * CH
ETQ = 1152
BIGF = 3.0e38
_SCP = pltpu.CompilerParams(use_tc_tiling_on_sc=False,
                            needs_layout_passes=False)


def _rsqrt_nr(v):
    i = lax.bitcast_convert_type(v, jnp.int32)
    i = 0x5F3759DF - lax.shift_right_logical(i, 1)
    y = lax.bitcast_convert_type(i, jnp.float32)
    for _ in range(3):
        y = y * (1.5 - 0.5 * v * y * y)
    return y


def _silu(z):
    return z / (1.0 + jnp.exp(-z))


def _make_mesh():
    return plsc.VectorSubcoreMesh(core_axis_name="c", subcore_axis_name="s",
                                  num_cores=NC, num_subcores=NS)


def _t1_body(x_ref, pos_ref, wx_ref, wr_ref, a_ref, b_ref, *, n, np_rows):
    b = jnp.dot(pos_ref[...], wr_ref[...], preferred_element_type=jnp.float32)
    a = jnp.dot(x_ref[...], wx_ref[...], preferred_element_type=jnp.float32) + b
    zpad = jnp.zeros((np_rows - n, D_OUT), jnp.float32)
    a_ref[...] = jnp.concatenate([a, zpad], axis=0)
    b_ref[...] = jnp.concatenate([b, zpad], axis=0)


def _t4_body(u_ref, d_ref, o_ref, *, n):
    u = u_ref[0, 0:n, :] + u_ref[1, 0:n, :]
    d = d_ref[0, 0:n] + d_ref[1, 0:n]
    o_ref[...] = u / (d[:, None] + 1e-16)


def _build_s1(e_pad, et, cpt):
    mesh = _make_mesh()

    @functools.partial(
        pl.kernel,
        out_type=(
            jax.ShapeDtypeStruct((e_pad, D_OUT), jnp.float32),
            jax.ShapeDtypeStruct((NW, 4 * L), jnp.float32),
        ),
        mesh=mesh,
        compiler_params=_SCP,
        scratch_types=(
            [pltpu.VMEM((cpt, CH), jnp.int32)] * 2
            + [pltpu.VMEM((CH, D_OUT), jnp.float32)] * (3 * NB)
            + [pltpu.VMEM((4 * L,), jnp.float32)]
            + [pltpu.SemaphoreType.DMA] * (3 * NB)
        ),
    )
    def s1(a_t, b_t, src_g, dst_g, l_g, part_g, *refs):
        idxs2, idxd2 = refs[0], refs[1]
        ras = refs[2:2 + NB]
        rbs = refs[2 + NB:2 + 2 * NB]
        lbs = refs[2 + 2 * NB:2 + 3 * NB]
        pbuf = refs[2 + 3 * NB]
        sas = refs[3 + 3 * NB:3 + 4 * NB]
        sbs = refs[3 + 4 * NB:3 + 5 * NB]
        sls = refs[3 + 5 * NB:3 + 6 * NB]

        cid = lax.axis_index("c")
        sid = lax.axis_index("s")
        wid = sid * NC + cid
        base = wid * et

        pltpu.sync_copy(src_g.at[wid], idxs2)
        pltpu.sync_copy(dst_g.at[wid], idxd2)

        def gissue(c, b):
            pltpu.async_copy(a_t.at[idxs2.at[c]], ras[b], sas[b])
            pltpu.async_copy(b_t.at[idxd2.at[c]], rbs[b], sbs[b])

        for b in range(NB):
            gissue(b, b)

        def outer(cg, carry):
            car = carry
            for b in range(NB):
                c = cg * NB + b
                pltpu.make_async_copy(a_t.at[idxs2.at[c]], ras[b],
                                      sas[b]).wait()
                pltpu.make_async_copy(b_t.at[idxd2.at[c]], rbs[b],
                                      sbs[b]).wait()

                @pl.when(cg > 0)
                def _():
                    pltpu.make_async_copy(
                        lbs[b], l_g.at[pl.ds(base + (c - NB) * CH, CH)],
                        sls[b]).wait()

                ra, rb, lb = ras[b], rbs[b], lbs[b]

                def edge(e, car2):
                    s0, s1v, q0, q1 = car2
                    l0 = ra[e, pl.ds(0, L)] - rb[e, pl.ds(0, L)]
                    l1 = ra[e, pl.ds(L, L)] - rb[e, pl.ds(L, L)]
                    lb[e, pl.ds(0, L)] = l0
                    lb[e, pl.ds(L, L)] = l1
                    return (s0 + l0, s1v + l1, q0 + l0 * l0, q1 + l1 * l1)

                car = lax.fori_loop(0, CH, edge, car, unroll=4)
                pltpu.async_copy(lbs[b], l_g.at[pl.ds(base + c * CH, CH)],
                                 sls[b])

                @pl.when(c + NB < cpt)
                def _():
                    gissue(c + NB, b)
            return car

        z = jnp.zeros((L,), jnp.float32)
        s0, s1v, q0, q1 = lax.fori_loop(0, cpt // NB, outer, (z, z, z, z))
        for b in range(NB):
            c = cpt - NB + b
            pltpu.make_async_copy(lbs[b], l_g.at[pl.ds(base + c * CH, CH)],
                                  sls[b]).wait()
        pbuf[pl.ds(0, L)] = s0
        pbuf[pl.ds(L, L)] = s1v
        pbuf[pl.ds(2 * L, L)] = q0
        pbuf[pl.ds(3 * L, L)] = q1
        pltpu.sync_copy(pbuf, part_g.at[wid])

    return s1


def _build_s2(e_pad, et, cpt2, e_tot):
    mesh = _make_mesh()
    inv_e = 1.0 / float(e_tot)

    @functools.partial(
        pl.kernel,
        out_type=(
            jax.ShapeDtypeStruct((e_pad, D_OUT), jnp.float32),
            jax.ShapeDtypeStruct((NW, et), jnp.float32),
            jax.ShapeDtypeStruct((NW, L), jnp.float32),
        ),
        mesh=mesh,
        compiler_params=_SCP,
        scratch_types=(
            [pltpu.VMEM((NW, 4 * L), jnp.float32),
             pltpu.VMEM((4 * L,), jnp.float32),
             pltpu.VMEM((2 * L,), jnp.float32),
             pltpu.VMEM((L,), jnp.float32)]
            + [pltpu.VMEM((CHL, D_OUT), jnp.float32)] * (2 * NB2)
            + [pltpu.VMEM((CHL,), jnp.float32)] * NB2
            + [pltpu.SemaphoreType.DMA] * (3 * NB2)
        ),
    )
    def s2(part_g, l_g, wg_g, prm_g, h_g, g_g, part2_g, *refs):
        pall, prmb, wgb, p2buf = refs[0], refs[1], refs[2], refs[3]
        lbs = refs[4:4 + NB2]
        hbs = refs[4 + NB2:4 + 2 * NB2]
        gbs = refs[4 + 2 * NB2:4 + 3 * NB2]
        sls = refs[4 + 3 * NB2:4 + 4 * NB2]
        shs = refs[4 + 4 * NB2:4 + 5 * NB2]
        sgs = refs[4 + 5 * NB2:4 + 6 * NB2]

        cid = lax.axis_index("c")
        sid = lax.axis_index("s")
        wid = sid * NC + cid
        base = wid * et

        for b in range(NB2):
            pltpu.async_copy(l_g.at[pl.ds(base + b * CHL, CHL)], lbs[b],
                             sls[b])

        pltpu.sync_copy(part_g, pall)
        pltpu.sync_copy(prm_g, prmb)
        pltpu.sync_copy(wg_g, wgb)

        def red(r, car):
            c0, c1, c2, c3 = car
            return (c0 + pall[r, pl.ds(0, L)],
                    c1 + pall[r, pl.ds(L, L)],
                    c2 + pall[r, pl.ds(2 * L, L)],
                    c3 + pall[r, pl.ds(3 * L, L)])

        z = jnp.zeros((L,), jnp.float32)
        sl0, sl1, sq0, sq1 = lax.fori_loop(0, NW, red, (z, z, z, z))
        m0 = sl0 * inv_e
        m1 = sl1 * inv_e
        v0 = sq0 * inv_e - m0 * m0
        v1 = sq1 * inv_e - m1 * m1
        r0 = _rsqrt_nr(v0 + 1e-5)
        r1 = _rsqrt_nr(v1 + 1e-5)
        s1a = prmb[pl.ds(0, L)] * r0
        s1b = prmb[pl.ds(L, L)] * r1
        t1a = prmb[pl.ds(2 * L, L)] - m0 * s1a
        t1b = prmb[pl.ds(3 * L, L)] - m1 * s1b
        li = lax.iota(jnp.int32, L)
        wglo = wgb[pl.ds(0, L)]
        wghi = wgb[pl.ds(L, L)]

        def outer(cg, carry):
            car = carry
            for b in range(NB2):
                c = cg * NB2 + b
                be = base + c * CHL
                pltpu.make_async_copy(l_g.at[pl.ds(be, CHL)], lbs[b],
                                      sls[b]).wait()

                @pl.when(cg > 0)
                def _():
                    pb = base + (c - NB2) * CHL
                    pltpu.make_async_copy(hbs[b], h_g.at[pl.ds(pb, CHL)],
                                          shs[b]).wait()
                    pltpu.make_async_copy(
                        gbs[b], g_g.at[wid, pl.ds(pb - base, CHL)],
                        sgs[b]).wait()

                lb, hb, gb = lbs[b], hbs[b], gbs[b]

                def kgroup(k, car2):
                    gs, gq, gmn, gmx = car2
                    gv = jnp.zeros((L,), jnp.float32)
                    for j in range(L):
                        e = k * L + j
                        z0 = lb[e, pl.ds(0, L)] * s1a + t1a
                        z1 = lb[e, pl.ds(L, L)] * s1b + t1b
                        h0 = _silu(z0)
                        h1 = _silu(z1)
                        hb[e, pl.ds(0, L)] = h0
                        hb[e, pl.ds(L, L)] = h1
                        s = jnp.sum(h0 * wglo + h1 * wghi)
                        gv = jnp.where(li == j, s, gv)
                    gb[pl.ds(k * L, L)] = gv
                    valid = (be + k * L + li) < e_tot
                    gz = jnp.where(valid, gv, 0.0)
                    return (gs + gz, gq + gz * gz,
                            jnp.minimum(gmn, jnp.where(valid, gv, BIGF)),
                            jnp.maximum(gmx, jnp.where(valid, gv, -BIGF)))

                car = lax.fori_loop(0, CHL // L, kgroup, car)

                @pl.when(c + NB2 < cpt2)
                def _():
                    nbo = base + (c + NB2) * CHL
                    pltpu.async_copy(l_g.at[pl.ds(nbo, CHL)], lbs[b], sls[b])
                pltpu.async_copy(hb, h_g.at[pl.ds(be, CHL)], shs[b])
                pltpu.async_copy(gb, g_g.at[wid, pl.ds(c * CHL, CHL)],
                                 sgs[b])
            return car

        gs, gq, gmn, gmx = lax.fori_loop(
            0, cpt2 // NB2, outer,
            (z, z, jnp.full((L,), BIGF), jnp.full((L,), -BIGF)))
        for b in range(NB2):
            pb = base + (cpt2 - NB2 + b) * CHL
            pltpu.make_async_copy(hbs[b], h_g.at[pl.ds(pb, CHL)],
                                  shs[b]).wait()
            pltpu.make_async_copy(gbs[b], g_g.at[wid, pl.ds(pb - base, CHL)],
                                  sgs[b]).wait()
        row = jnp.where(li == 0, jnp.sum(gs),
                        jnp.where(li == 1, jnp.sum(gq),
                                  jnp.where(li == 2, jnp.min(gmn),
                                            jnp.where(li == 3, jnp.max(gmx),
                                                      0.0))))
        p2buf[...] = row
        pltpu.sync_copy(p2buf, part2_g.at[wid])

    return s2


def _build_s3(e_pad, et, cpt, cpt3, e_tot, na):
    mesh = _make_mesh()
    inv_e = 1.0 / float(e_tot)
    rows_per_tile = na // NS
    sub = CHS // CH

    @functools.partial(
        pl.kernel,
        out_type=(
            jax.ShapeDtypeStruct((NC, na, D_OUT), jnp.float32),
            jax.ShapeDtypeStruct((NC, na), jnp.float32),
        ),
        mesh=mesh,
        compiler_params=_SCP,
        scratch_types=(
            [pltpu.VMEM((NW, L), jnp.float32),
             pltpu.VMEM((L,), jnp.float32),
             pltpu.VMEM((cpt, CH), jnp.int32),
             pltpu.VMEM((et,), jnp.float32),
             pltpu.VMEM((et,), jnp.float32),
             pltpu.VMEM((CH,), jnp.float32)]
            + [pltpu.VMEM((CHS, D_OUT), jnp.float32)] * (2 * NB)
            + [pltpu.VMEM_SHARED((na, D_OUT), jnp.float32),
               pltpu.VMEM_SHARED((na,), jnp.float32)]
            + [pltpu.SemaphoreType.DMA] * (2 * NB + 1)
        ),
    )
    def s3(part2_g, h_g, g_g, dst_g, prm2_g, u_acc_g, d_acc_g, *refs):
        p2all, prm2b, idxd2, gall, wall, zdbuf = refs[0:6]
        hbs = refs[6:6 + NB]
        vbs = refs[6 + NB:6 + 2 * NB]
        uacc, dacc = refs[6 + 2 * NB], refs[7 + 2 * NB]
        shs = refs[8 + 2 * NB:8 + 3 * NB]
        sus = refs[8 + 3 * NB:8 + 4 * NB]
        sds = refs[8 + 4 * NB]

        cid = lax.axis_index("c")
        sid = lax.axis_index("s")
        wid = sid * NC + cid
        base = wid * et

        for b in range(NB):
            pltpu.async_copy(h_g.at[pl.ds(base + b * CHS, CHS)], hbs[b],
                             shs[b])
        pltpu.sync_copy(part2_g, p2all)
        pltpu.sync_copy(prm2_g, prm2b)
        pltpu.sync_copy(dst_g.at[wid], idxd2)
        pltpu.sync_copy(g_g.at[wid], gall)

        p2v = prm2b[pl.ds(0, L)]
        li = lax.iota(jnp.int32, L)
        is_sum = li < 2
        is_min = li == 2

        def red(r, acc_v):
            rrow = p2all[r, pl.ds(0, L)]
            return jnp.where(is_sum, acc_v + rrow,
                             jnp.where(is_min, jnp.minimum(acc_v, rrow),
                                       jnp.maximum(acc_v, rrow)))

        accv = lax.fori_loop(1, NW, red, p2all[0, pl.ds(0, L)])
        gsum = jnp.sum(jnp.where(li == 0, accv, 0.0))
        gsq = jnp.sum(jnp.where(li == 1, accv, 0.0))
        gmn = jnp.sum(jnp.where(li == 2, accv, 0.0))
        gmx = jnp.sum(jnp.where(li == 3, accv, 0.0))
        m2 = gsum * inv_e
        v2 = gsq * inv_e - m2 * m2
        rsv = _rsqrt_nr(jnp.full((L,), v2 + 1e-5))
        s2v = rsv * p2v[0]
        t2v = p2v[1] - m2 * s2v
        ze = jnp.where(li == 0, gmn, gmx) * s2v + t2v
        cmax = jnp.max(_silu(ze))
        s2s = jnp.sum(jnp.where(li == 0, s2v, 0.0))
        t2s = jnp.sum(jnp.where(li == 0, t2v, 0.0))

        def wstage(k, car):
            gv = gall[pl.ds(k * L, L)]
            gate = _silu(gv * s2s + t2s)
            wv = jnp.exp(gate - cmax)
            wall[pl.ds(k * L, L)] = jnp.where(base + k * L + li < e_tot,
                                              wv, 0.0)
            return car

        lax.fori_loop(0, et // L, wstage, 0, unroll=4)

        zv = jnp.zeros((L,), jnp.float32)
        zstage = vbs[0]

        def zrow(r, car):
            zstage[r, pl.ds(0, L)] = zv
            zstage[r, pl.ds(L, L)] = zv
            return car

        lax.fori_loop(0, CH, zrow, 0)
        for k in range(CH // L):
            zdbuf[pl.ds(k * L, L)] = zv
        for j in range(rows_per_tile // CH):
            pltpu.sync_copy(zstage.at[pl.ds(0, CH)],
                            uacc.at[pl.ds(sid * rows_per_tile + j * CH, CH)])
            pltpu.sync_copy(zdbuf,
                            dacc.at[pl.ds(sid * rows_per_tile + j * CH, CH)])
        plsc.subcore_barrier()

        def outer(cg, car):
            for b in range(NB):
                c = cg * NB + b
                be = base + c * CHS
                pltpu.make_async_copy(h_g.at[pl.ds(be, CHS)], hbs[b],
                                      shs[b]).wait()

                @pl.when(cg > 0)
                def _():
                    for j in range(sub):
                        pltpu.make_async_copy(
                            vbs[b].at[pl.ds(j * CH, CH)],
                            uacc.at[idxd2.at[(c - NB) * sub + j]],
                            sus[b]).wait()

                hb, vb = hbs[b], vbs[b]

                def group(k, car2):
                    wv = wall[pl.ds(c * CHS + k * L, L)]
                    for j in range(L):
                        e = k * L + j
                        w = wv[j]
                        vb[e, pl.ds(0, L)] = hb[e, pl.ds(0, L)] * w
                        vb[e, pl.ds(L, L)] = hb[e, pl.ds(L, L)] * w
                    return car2

                lax.fori_loop(0, CHS // L, group, 0)

                @pl.when(c + NB < cpt3)
                def _():
                    nbo = base + (c + NB) * CHS
                    pltpu.async_copy(h_g.at[pl.ds(nbo, CHS)], hbs[b], shs[b])

                for j in range(sub):
                    pltpu.async_copy(
                        vb.at[pl.ds(j * CH, CH)],
                        uacc.at[idxd2.at[c * sub + j]],
                        sus[b], add=True)
                    pltpu.async_copy(
                        wall.at[pl.ds(c * CHS + j * CH, CH)],
                        dacc.at[idxd2.at[c * sub + j]],
                        sds, add=True)
            return car

        lax.fori_loop(0, cpt3 // NB, outer, 0)
        for b in range(NB):
            c = cpt3 - NB + b
            for j in range(sub):
                pltpu.make_async_copy(
                    vbs[b].at[pl.ds(j * CH, CH)],
                    uacc.at[idxd2.at[c * sub + j]],
                    sus[b]).wait()
        for c in range(cpt3):
            for j in range(sub):
                pltpu.make_async_copy(
                    wall.at[pl.ds(c * CHS + j * CH, CH)],
                    dacc.at[idxd2.at[c * sub + j]],
                    sds).wait()
        plsc.subcore_barrier()
        pltpu.sync_copy(
            uacc.at[pl.ds(sid * rows_per_tile, rows_per_tile)],
            u_acc_g.at[cid, pl.ds(sid * rows_per_tile, rows_per_tile)])
        pltpu.sync_copy(
            dacc.at[pl.ds(sid * rows_per_tile, rows_per_tile)],
            d_acc_g.at[cid, pl.ds(sid * rows_per_tile, rows_per_tile)])

    return s3


def kernel(x, pos, edge_index, W_local, b_local, bn_local_gamma,
           bn_local_beta, W_gate, b_gate, bn_gate_gamma, bn_gate_beta):
    n = x.shape[0]
    e_tot = edge_index.shape[1] + n
    et = -(-e_tot // (NW * ETQ)) * ETQ
    cpt = et // CH
    cpt2 = et // CHL
    cpt3 = et // CHS
    e_pad = et * NW
    np_rows = ((n + 16) + 7) // 8 * 8
    na = -(-np_rows // (NS * CH)) * NS * CH

    idt = edge_index.dtype
    loop = jnp.arange(n, dtype=idt)
    pad = e_pad - e_tot
    ei3 = jnp.concatenate(
        [edge_index, jnp.broadcast_to(loop, (2, n)),
         jnp.full((2, pad), n, idt)], axis=1).reshape(2, NW, cpt, CH)
    src3 = ei3[0]
    dst3 = ei3[1]
    wx = W_local[:D_IN]
    wr = W_local[D_IN:]
    prm1 = jnp.concatenate([bn_local_gamma, bn_local_beta])
    wg = W_gate[:, 0]
    prm2 = jnp.concatenate(
        [bn_gate_gamma, bn_gate_beta, jnp.zeros((14,), jnp.float32)])

    a_t, b_t = pl.pallas_call(
        functools.partial(_t1_body, n=n, np_rows=np_rows),
        out_shape=(
            jax.ShapeDtypeStruct((np_rows, D_OUT), jnp.float32),
            jax.ShapeDtypeStruct((np_rows, D_OUT), jnp.float32),
        ),
    )(x, pos, wx, wr)

    l_g, part = _build_s1(e_pad, et, cpt)(a_t, b_t, src3, dst3)
    h_g, g2, part2 = _build_s2(e_pad, et, cpt2, e_tot)(part, l_g, wg, prm1)
    u_p, d_p = _build_s3(e_pad, et, cpt, cpt3, e_tot, na)(
        part2, h_g, g2, dst3, prm2)

    out = pl.pallas_call(
        functools.partial(_t4_body, n=n),
        out_shape=jax.ShapeDtypeStruct((n, D_OUT), jnp.float32),
    )(u_p, d_p)
    return out
